# baseline (speedup 1.0000x reference)
import functools

import numpy as np
import jax
import jax.numpy as jnp
from jax import lax
from jax.experimental import pallas as pl
from jax.experimental.pallas import tpu as pltpu
from jax.experimental.pallas import tpu_sc as plsc

_N_FREQ = 10
_SINE_DIM = _N_FREQ * 2
_D_EMB = 20

_FREQS = np.array(
    [[1.0 / 1000.0 ** (2.0 * i / _N_FREQ)] * 2 for i in range(1, _N_FREQ + 1)],
    dtype=np.float32,
).reshape(-1, 1)
_SHIFTS = np.array(
    [[0.0, np.pi / 2.0] for _ in range(1, _N_FREQ + 1)], dtype=np.float32
).reshape(-1, 1)

_NC = 2
_NS = 16
_NW = _NC * _NS
_CHUNK = 128


def _sc_gather_cols(table2, rix3, B):
    bpw = B // _NW
    nch = bpw // _CHUNK
    mesh = plsc.VectorSubcoreMesh(core_axis_name="c", subcore_axis_name="s")

    @functools.partial(
        pl.kernel,
        mesh=mesh,
        out_type=jax.ShapeDtypeStruct((_D_EMB, B), jnp.float32),
        scratch_types=[
            pltpu.VMEM((nch, _CHUNK), jnp.int32),
            pltpu.VMEM((_D_EMB * bpw,), jnp.float32),
            pltpu.SemaphoreType.DMA,
        ],
    )
    def gather_kernel(rix_hbm, table_hbm, out_hbm, idx_v, rows_v, sem):
        wid = lax.axis_index("s") * _NC + lax.axis_index("c")
        pltpu.sync_copy(rix_hbm.at[wid], idx_v)

        @pl.loop(0, _D_EMB)
        def fire(d):
            for c in range(nch):
                pltpu.async_copy(
                    table_hbm.at[d, 0].at[idx_v.at[c]],
                    rows_v.at[pl.ds(d * bpw + c * _CHUNK, _CHUNK)],
                    sem,
                )

        pltpu.make_async_copy(
            table_hbm.at[0, 0].at[pl.ds(0, _D_EMB * bpw)], rows_v, sem
        ).wait()

        puts = [
            pltpu.async_copy(
                rows_v.at[pl.ds(d * bpw, bpw)],
                out_hbm.at[d, pl.ds(wid * bpw, bpw)],
                sem,
            )
            for d in range(_D_EMB)
        ]
        for p in puts:
            p.wait()

    return gather_kernel(rix3, table2)


def _tc_combine_t(coords_row, w0, b0c, gt):
    B = coords_row.shape[1]
    blk = min(B, 16384)
    grid = B // blk
    fs = jnp.asarray(_FREQS)
    sh = jnp.asarray(_SHIFTS)

    def body(c_ref, w_ref, b_ref, f_ref, s_ref, g_ref, o_ref):
        c = c_ref[...]
        x = f_ref[...] * c + s_ref[...]
        e = jnp.sin(x)
        h = jnp.dot(w_ref[...], e, preferred_element_type=jnp.float32)
        h = jax.nn.sigmoid(h + b_ref[...])
        o_ref[...] = jnp.sum(h * g_ref[...], axis=0, keepdims=True)

    return pl.pallas_call(
        body,
        grid=(grid,),
        in_specs=[
            pl.BlockSpec((1, blk), lambda i: (0, i)),
            pl.BlockSpec((_D_EMB, _SINE_DIM), lambda i: (0, 0)),
            pl.BlockSpec((_D_EMB, 1), lambda i: (0, 0)),
            pl.BlockSpec((_SINE_DIM, 1), lambda i: (0, 0)),
            pl.BlockSpec((_SINE_DIM, 1), lambda i: (0, 0)),
            pl.BlockSpec((_D_EMB, blk), lambda i: (0, i)),
        ],
        out_specs=pl.BlockSpec((1, blk), lambda i: (0, i)),
        out_shape=jax.ShapeDtypeStruct((1, B), jnp.float32),
    )(coords_row, w0, b0c, fs, sh, gt)


def kernel(coordinates, region_ix, W0, b0, weight1):
    B = coordinates.shape[0]
    rix = region_ix.astype(jnp.int32)
    n_regions = weight1.shape[0]
    table2 = weight1.transpose(1, 2, 0)
    rix3 = rix.reshape(_NW, B // _NW // _CHUNK, _CHUNK)
    gt = _sc_gather_cols(table2, rix3, B)
    out_row = _tc_combine_t(
        coordinates.reshape(1, B),
        W0,
        b0.reshape(-1, 1),
        gt,
    )
    return out_row.reshape(B, 1)


# speedup vs baseline: 1.0041x; 1.0041x over previous
import functools

import numpy as np
import jax
import jax.numpy as jnp
from jax import lax
from jax.experimental import pallas as pl
from jax.experimental.pallas import tpu as pltpu
from jax.experimental.pallas import tpu_sc as plsc

_N_FREQ = 10
_SINE_DIM = _N_FREQ * 2
_D_EMB = 20

_FREQS = np.array(
    [[1.0 / 1000.0 ** (2.0 * i / _N_FREQ)] * 2 for i in range(1, _N_FREQ + 1)],
    dtype=np.float32,
).reshape(-1, 1)
_SHIFTS = np.array(
    [[0.0, np.pi / 2.0] for _ in range(1, _N_FREQ + 1)], dtype=np.float32
).reshape(-1, 1)

_NC = 2
_NS = 16
_NW = _NC * _NS
_CHUNK = 128


def _sc_gather_cols(table2, rix3, B):
    bpw = B // _NW
    nch = bpw // _CHUNK
    mesh = plsc.VectorSubcoreMesh(core_axis_name="c", subcore_axis_name="s")

    @functools.partial(
        pl.kernel,
        mesh=mesh,
        out_type=jax.ShapeDtypeStruct((_D_EMB, B), jnp.float32),
        scratch_types=[
            pltpu.VMEM((nch, _CHUNK), jnp.int32),
            pltpu.VMEM((_D_EMB * bpw,), jnp.float32),
            pltpu.SemaphoreType.DMA,
        ],
    )
    def gather_kernel(rix_hbm, table_hbm, out_hbm, idx_v, rows_v, sem):
        wid = lax.axis_index("s") * _NC + lax.axis_index("c")
        pltpu.sync_copy(rix_hbm.at[wid], idx_v)

        @pl.loop(0, _D_EMB)
        def fire(d):
            for c in range(nch):
                pltpu.async_copy(
                    table_hbm.at[d, 0].at[idx_v.at[c]],
                    rows_v.at[pl.ds(d * bpw + c * _CHUNK, _CHUNK)],
                    sem,
                )

        pltpu.make_async_copy(
            table_hbm.at[0, 0].at[pl.ds(0, _D_EMB * bpw)], rows_v, sem
        ).wait()

        puts = [
            pltpu.async_copy(
                rows_v.at[pl.ds(d * bpw, bpw)],
                out_hbm.at[d, pl.ds(wid * bpw, bpw)],
                sem,
            )
            for d in range(_D_EMB)
        ]
        for p in puts:
            p.wait()

    return gather_kernel(rix3, table2)


def _tc_combine_t(coords_row, w0, b0c, gt):
    B = coords_row.shape[1]
    blk = min(B, 8192)
    grid = B // blk
    fs = jnp.asarray(_FREQS)
    sh = jnp.asarray(_SHIFTS)

    def body(c_ref, w_ref, b_ref, f_ref, s_ref, g_ref, o_ref):
        c = c_ref[...]
        x = f_ref[...] * c + s_ref[...]
        e = jnp.sin(x)
        h = jnp.dot(w_ref[...], e, preferred_element_type=jnp.float32)
        h = jax.nn.sigmoid(h + b_ref[...])
        o_ref[...] = jnp.sum(h * g_ref[...], axis=0, keepdims=True)

    return pl.pallas_call(
        body,
        grid=(grid,),
        in_specs=[
            pl.BlockSpec((1, blk), lambda i: (0, i)),
            pl.BlockSpec((_D_EMB, _SINE_DIM), lambda i: (0, 0)),
            pl.BlockSpec((_D_EMB, 1), lambda i: (0, 0)),
            pl.BlockSpec((_SINE_DIM, 1), lambda i: (0, 0)),
            pl.BlockSpec((_SINE_DIM, 1), lambda i: (0, 0)),
            pl.BlockSpec((_D_EMB, blk), lambda i: (0, i)),
        ],
        out_specs=pl.BlockSpec((1, blk), lambda i: (0, i)),
        out_shape=jax.ShapeDtypeStruct((1, B), jnp.float32),
    )(coords_row, w0, b0c, fs, sh, gt)


def kernel(coordinates, region_ix, W0, b0, weight1):
    B = coordinates.shape[0]
    rix = region_ix.astype(jnp.int32)
    table2 = weight1.transpose(1, 2, 0)
    rix3 = rix.reshape(_NW, B // _NW // _CHUNK, _CHUNK)
    gt = _sc_gather_cols(table2, rix3, B)
    out_row = _tc_combine_t(
        coordinates.reshape(1, B),
        W0,
        b0.reshape(-1, 1),
        gt,
    )
    return out_row.reshape(B, 1)
